# baseline (speedup 1.0000x reference)
import functools

import jax
import jax.numpy as jnp
from jax.experimental import pallas as pl

_B, _VSC, _VST, _H = 2, 256, 256, 128
_EPS = 1e-5
_BI = 16


def _layernorm(x, g, b):
    mu = jnp.mean(x, axis=-1, keepdims=True)
    var = jnp.mean((x - mu) ** 2, axis=-1, keepdims=True)
    return (x - mu) / jnp.sqrt(var + _EPS) * g + b


def _proj_kernel(hsc_ref, hst_ref, wsc_ref, wst_ref, bsc_ref, bst_ref,
                 osc_ref, ost_ref):
    hsc = hsc_ref[...].reshape(_B * _VSC, _H)
    hst = hst_ref[...].reshape(_B * _VST, _H)
    osc = jnp.dot(hsc, wsc_ref[...], preferred_element_type=jnp.float32)
    ost = jnp.dot(hst, wst_ref[...], preferred_element_type=jnp.float32)
    osc_ref[...] = (osc + bsc_ref[...]).reshape(_B, _VSC, 6 * _H)
    ost_ref[...] = (ost + bst_ref[...]).reshape(_B, _VST, 6 * _H)


def _edge_kernel(e_ref, arow_ref, bcol_ref, vrow_ref, wc_ref, avg_ref,
                 ge_ref, be_ref, *rest, ncols, with_col):
    if with_col:
        vcol_ref, eout_ref, row_ref, col_ref = rest
    else:
        eout_ref, row_ref = rest
    x = e_ref[0]
    xm = jnp.dot(x.reshape(_BI * ncols, _H), wc_ref[...],
                 preferred_element_type=jnp.float32)
    e_new = (xm.reshape(_BI, ncols, _H)
             + arow_ref[0][:, None, :] + bcol_ref[0][None, :, :])
    g = 1.0 / (1.0 + jnp.exp(-e_new))
    row_ref[0] = jnp.sum(g * vrow_ref[0][None, :, :], axis=1)
    if with_col:
        part = jnp.sum(g * vcol_ref[0][:, None, :], axis=0)

        @pl.when(pl.program_id(1) == 0)
        def _():
            col_ref[0] = part

        @pl.when(pl.program_id(1) != 0)
        def _():
            col_ref[0] += part

    e2 = e_new.reshape(_BI * ncols, _H)
    hi = jax.lax.Precision.HIGHEST
    mu = jnp.dot(e2, avg_ref[...], precision=hi,
                 preferred_element_type=jnp.float32)
    msq = jnp.dot(e2 * e2, avg_ref[...], precision=hi,
                  preferred_element_type=jnp.float32)
    scale = jax.lax.rsqrt(msq - mu * mu + _EPS) * ge_ref[0]
    ln = (e2 - mu) * scale + be_ref[0]
    eout_ref[0] = x + jnp.maximum(ln, 0.0).reshape(_BI, ncols, _H)


def _node_kernel(uhsc_ref, uhst_ref, st2sc_ref, sc2sc_ref, sc2st_ref,
                 st2st_ref, hsc_ref, hst_ref, gh_ref, bh_ref,
                 osc_ref, ost_ref):
    xsc = uhsc_ref[...] + st2sc_ref[...] + sc2sc_ref[...]
    xst = uhst_ref[...] + sc2st_ref[...] + st2st_ref[...]
    osc_ref[...] = hsc_ref[...] + jnp.maximum(
        _layernorm(xsc, gh_ref[0], bh_ref[0]), 0.0)
    ost_ref[...] = hst_ref[...] + jnp.maximum(
        _layernorm(xst, gh_ref[0], bh_ref[0]), 0.0)


def _edge_call(e, proj_row, proj_st_or_sc, arow_idx, bcol_idx, vrow_idx,
               wc, ge, be, nrows, ncols, with_col, vcol_idx=None):
    nblk = nrows // _BI
    vec = lambda v: v.reshape(1, _H)
    small = pl.BlockSpec((1, _H), lambda b, i: (0, 0))
    full_col = lambda idx: pl.BlockSpec((1, ncols, _H),
                                        lambda b, i, idx=idx: (b, 0, idx))
    row_blk = lambda idx: pl.BlockSpec((1, _BI, _H),
                                       lambda b, i, idx=idx: (b, i, idx))
    in_specs = [
        pl.BlockSpec((1, _BI, ncols, _H), lambda b, i: (b, i, 0, 0)),
        row_blk(arow_idx),
        full_col(bcol_idx),
        full_col(vrow_idx),
        pl.BlockSpec((_H, _H), lambda b, i: (0, 0)),
        pl.BlockSpec((_H, _H), lambda b, i: (0, 0)),
        small, small,
    ]
    out_shapes = [
        jax.ShapeDtypeStruct((_B, nrows, ncols, _H), jnp.float32),
        jax.ShapeDtypeStruct((_B, nrows, _H), jnp.float32),
    ]
    out_specs = [
        pl.BlockSpec((1, _BI, ncols, _H), lambda b, i: (b, i, 0, 0)),
        pl.BlockSpec((1, _BI, _H), lambda b, i: (b, i, 0)),
    ]
    args = [e, proj_row, proj_st_or_sc, proj_st_or_sc, wc,
            jnp.full((_H, _H), 1.0 / _H, jnp.float32), vec(ge), vec(be)]
    if with_col:
        in_specs.append(row_blk(vcol_idx))
        args.append(proj_row)
        out_shapes.append(jax.ShapeDtypeStruct((_B, ncols, _H), jnp.float32))
        out_specs.append(pl.BlockSpec((1, ncols, _H), lambda b, i: (b, 0, 0)))
    return pl.pallas_call(
        functools.partial(_edge_kernel, ncols=ncols, with_col=with_col),
        grid=(_B, nblk),
        in_specs=in_specs,
        out_specs=out_specs,
        out_shape=out_shapes,
    )(*args)


def kernel(h_sc, h_st, bi_e, bi_graph, sc_e, sc_graph, st_e, st_graph,
           params):
    p = params
    wsc = jnp.concatenate([p["U1"]["w"], p["V1"]["w"], p["W1"]["w"],
                           p["bi_A"]["w"], p["sc_A"]["w"], p["sc_B"]["w"]],
                          axis=0).T
    wst = jnp.concatenate([p["U2"]["w"], p["V2"]["w"], p["W2"]["w"],
                           p["bi_B"]["w"], p["st_A"]["w"], p["st_B"]["w"]],
                          axis=0).T
    bsc = jnp.concatenate([p["U1"]["b"], p["V1"]["b"], p["W1"]["b"],
                           p["bi_A"]["b"] + p["bi_C"]["b"],
                           p["sc_A"]["b"] + p["sc_C"]["b"],
                           p["sc_B"]["b"]]).reshape(1, 6 * _H)
    bst = jnp.concatenate([p["U2"]["b"], p["V2"]["b"], p["W2"]["b"],
                           p["bi_B"]["b"],
                           p["st_A"]["b"] + p["st_C"]["b"],
                           p["st_B"]["b"]]).reshape(1, 6 * _H)

    proj_sc, proj_st = pl.pallas_call(
        _proj_kernel,
        out_shape=[jax.ShapeDtypeStruct((_B, _VSC, 6 * _H), jnp.float32),
                   jax.ShapeDtypeStruct((_B, _VST, 6 * _H), jnp.float32)],
    )(h_sc, h_st, wsc, wst, bsc, bst)

    ge, be = p["ln_e"]["g"], p["ln_e"]["b"]

    bi_e_out, h_st2sc, h_sc2st = _edge_call(
        bi_e, proj_sc, proj_st, arow_idx=3, bcol_idx=3, vrow_idx=1,
        wc=p["bi_C"]["w"].T, ge=ge, be=be,
        nrows=_VSC, ncols=_VST, with_col=True, vcol_idx=1)
    sc_e_out, h_sc2sc = _edge_call(
        sc_e, proj_sc, proj_sc, arow_idx=4, bcol_idx=5, vrow_idx=2,
        wc=p["sc_C"]["w"].T, ge=ge, be=be,
        nrows=_VSC, ncols=_VSC, with_col=False)
    st_e_out, h_st2st = _edge_call(
        st_e, proj_st, proj_st, arow_idx=4, bcol_idx=5, vrow_idx=2,
        wc=p["st_C"]["w"].T, ge=ge, be=be,
        nrows=_VST, ncols=_VST, with_col=False)

    full = pl.BlockSpec((_B, _VSC, _H), lambda i: (0, 0, 0))
    small = pl.BlockSpec((1, _H), lambda i: (0, 0))
    h_sc_out, h_st_out = pl.pallas_call(
        _node_kernel,
        grid=(1,),
        in_specs=[full, full, full, full, full, full, full, full,
                  small, small],
        out_specs=[full, full],
        out_shape=[jax.ShapeDtypeStruct((_B, _VSC, _H), jnp.float32),
                   jax.ShapeDtypeStruct((_B, _VST, _H), jnp.float32)],
    )(proj_sc, proj_st, h_st2sc, h_sc2sc, h_sc2st, h_st2st, h_sc, h_st,
      p["ln_h"]["g"].reshape(1, _H), p["ln_h"]["b"].reshape(1, _H))

    return (h_sc_out, h_st_out, bi_e_out, sc_e_out, st_e_out)


# speedup vs baseline: 1.8378x; 1.8378x over previous
import functools

import jax
import jax.numpy as jnp
from jax.experimental import pallas as pl

_B, _VSC, _VST, _H = 2, 256, 256, 128
_EPS = 1e-5
_BI = 16


def _layernorm(x, g, b):
    mu = jnp.mean(x, axis=-1, keepdims=True)
    var = jnp.mean((x - mu) ** 2, axis=-1, keepdims=True)
    return (x - mu) / jnp.sqrt(var + _EPS) * g + b


def _proj_kernel(hsc_ref, hst_ref, wsc_ref, wst_ref, bsc_ref, bst_ref,
                 osc_ref, ost_ref):
    hsc = hsc_ref[...].reshape(_B * _VSC, _H)
    hst = hst_ref[...].reshape(_B * _VST, _H)
    osc = jnp.dot(hsc, wsc_ref[...], preferred_element_type=jnp.float32)
    ost = jnp.dot(hst, wst_ref[...], preferred_element_type=jnp.float32)
    osc_ref[...] = (osc + bsc_ref[...]).reshape(_B, _VSC, 6 * _H)
    ost_ref[...] = (ost + bst_ref[...]).reshape(_B, _VST, 6 * _H)


def _edge_kernel(e_ref, arow_ref, bcol_ref, vrow_ref, wc_ref,
                 ge_ref, be_ref, *rest, ncols, with_col):
    if with_col:
        vcol_ref, eout_ref, row_ref, col_ref = rest
    else:
        eout_ref, row_ref = rest
    x = e_ref[0]
    xm = jnp.dot(x.reshape(_BI * ncols, _H), wc_ref[...],
                 preferred_element_type=jnp.float32)
    e_new = (xm.reshape(_BI, ncols, _H)
             + arow_ref[0][:, None, :] + bcol_ref[0][None, :, :])
    g = 1.0 / (1.0 + jnp.exp(-e_new))
    row_ref[0] = jnp.sum(g * vrow_ref[0][None, :, :], axis=1)
    if with_col:
        part = jnp.sum(g * vcol_ref[0][:, None, :], axis=0)

        @pl.when(pl.program_id(1) == 0)
        def _():
            col_ref[0] = part

        @pl.when(pl.program_id(1) != 0)
        def _():
            col_ref[0] += part

    ln = _layernorm(e_new, ge_ref[0], be_ref[0])
    eout_ref[0] = x + jnp.maximum(ln, 0.0)


def _node_kernel(uhsc_ref, uhst_ref, st2sc_ref, sc2sc_ref, sc2st_ref,
                 st2st_ref, hsc_ref, hst_ref, gh_ref, bh_ref,
                 osc_ref, ost_ref):
    xsc = uhsc_ref[...] + st2sc_ref[...] + sc2sc_ref[...]
    xst = uhst_ref[...] + sc2st_ref[...] + st2st_ref[...]
    osc_ref[...] = hsc_ref[...] + jnp.maximum(
        _layernorm(xsc, gh_ref[0], bh_ref[0]), 0.0)
    ost_ref[...] = hst_ref[...] + jnp.maximum(
        _layernorm(xst, gh_ref[0], bh_ref[0]), 0.0)


def _edge_call(e, proj_row, proj_st_or_sc, arow_idx, bcol_idx, vrow_idx,
               wc, ge, be, nrows, ncols, with_col, vcol_idx=None):
    nblk = nrows // _BI
    vec = lambda v: v.reshape(1, _H)
    small = pl.BlockSpec((1, _H), lambda b, i: (0, 0))
    full_col = lambda idx: pl.BlockSpec((1, ncols, _H),
                                        lambda b, i, idx=idx: (b, 0, idx))
    row_blk = lambda idx: pl.BlockSpec((1, _BI, _H),
                                       lambda b, i, idx=idx: (b, i, idx))
    in_specs = [
        pl.BlockSpec((1, _BI, ncols, _H), lambda b, i: (b, i, 0, 0)),
        row_blk(arow_idx),
        full_col(bcol_idx),
        full_col(vrow_idx),
        pl.BlockSpec((_H, _H), lambda b, i: (0, 0)),
        small, small,
    ]
    out_shapes = [
        jax.ShapeDtypeStruct((_B, nrows, ncols, _H), jnp.float32),
        jax.ShapeDtypeStruct((_B, nrows, _H), jnp.float32),
    ]
    out_specs = [
        pl.BlockSpec((1, _BI, ncols, _H), lambda b, i: (b, i, 0, 0)),
        pl.BlockSpec((1, _BI, _H), lambda b, i: (b, i, 0)),
    ]
    args = [e, proj_row, proj_st_or_sc, proj_st_or_sc, wc,
            vec(ge), vec(be)]
    if with_col:
        in_specs.append(row_blk(vcol_idx))
        args.append(proj_row)
        out_shapes.append(jax.ShapeDtypeStruct((_B, ncols, _H), jnp.float32))
        out_specs.append(pl.BlockSpec((1, ncols, _H), lambda b, i: (b, 0, 0)))
    return pl.pallas_call(
        functools.partial(_edge_kernel, ncols=ncols, with_col=with_col),
        grid=(_B, nblk),
        in_specs=in_specs,
        out_specs=out_specs,
        out_shape=out_shapes,
    )(*args)


def kernel(h_sc, h_st, bi_e, bi_graph, sc_e, sc_graph, st_e, st_graph,
           params):
    p = params
    wsc = jnp.concatenate([p["U1"]["w"], p["V1"]["w"], p["W1"]["w"],
                           p["bi_A"]["w"], p["sc_A"]["w"], p["sc_B"]["w"]],
                          axis=0).T
    wst = jnp.concatenate([p["U2"]["w"], p["V2"]["w"], p["W2"]["w"],
                           p["bi_B"]["w"], p["st_A"]["w"], p["st_B"]["w"]],
                          axis=0).T
    bsc = jnp.concatenate([p["U1"]["b"], p["V1"]["b"], p["W1"]["b"],
                           p["bi_A"]["b"] + p["bi_C"]["b"],
                           p["sc_A"]["b"] + p["sc_C"]["b"],
                           p["sc_B"]["b"]]).reshape(1, 6 * _H)
    bst = jnp.concatenate([p["U2"]["b"], p["V2"]["b"], p["W2"]["b"],
                           p["bi_B"]["b"],
                           p["st_A"]["b"] + p["st_C"]["b"],
                           p["st_B"]["b"]]).reshape(1, 6 * _H)

    proj_sc, proj_st = pl.pallas_call(
        _proj_kernel,
        out_shape=[jax.ShapeDtypeStruct((_B, _VSC, 6 * _H), jnp.float32),
                   jax.ShapeDtypeStruct((_B, _VST, 6 * _H), jnp.float32)],
    )(h_sc, h_st, wsc, wst, bsc, bst)

    ge, be = p["ln_e"]["g"], p["ln_e"]["b"]

    bi_e_out, h_st2sc, h_sc2st = _edge_call(
        bi_e, proj_sc, proj_st, arow_idx=3, bcol_idx=3, vrow_idx=1,
        wc=p["bi_C"]["w"].T, ge=ge, be=be,
        nrows=_VSC, ncols=_VST, with_col=True, vcol_idx=1)
    sc_e_out, h_sc2sc = _edge_call(
        sc_e, proj_sc, proj_sc, arow_idx=4, bcol_idx=5, vrow_idx=2,
        wc=p["sc_C"]["w"].T, ge=ge, be=be,
        nrows=_VSC, ncols=_VSC, with_col=False)
    st_e_out, h_st2st = _edge_call(
        st_e, proj_st, proj_st, arow_idx=4, bcol_idx=5, vrow_idx=2,
        wc=p["st_C"]["w"].T, ge=ge, be=be,
        nrows=_VST, ncols=_VST, with_col=False)

    full = pl.BlockSpec((_B, _VSC, _H), lambda i: (0, 0, 0))
    small = pl.BlockSpec((1, _H), lambda i: (0, 0))
    h_sc_out, h_st_out = pl.pallas_call(
        _node_kernel,
        grid=(1,),
        in_specs=[full, full, full, full, full, full, full, full,
                  small, small],
        out_specs=[full, full],
        out_shape=[jax.ShapeDtypeStruct((_B, _VSC, _H), jnp.float32),
                   jax.ShapeDtypeStruct((_B, _VST, _H), jnp.float32)],
    )(proj_sc, proj_st, h_st2sc, h_sc2sc, h_sc2st, h_st2st, h_sc, h_st,
      p["ln_h"]["g"].reshape(1, _H), p["ln_h"]["b"].reshape(1, _H))

    return (h_sc_out, h_st_out, bi_e_out, sc_e_out, st_e_out)


# speedup vs baseline: 2.4165x; 1.3149x over previous
import functools

import jax
import jax.numpy as jnp
from jax.experimental import pallas as pl

_B, _VSC, _VST, _H = 2, 256, 256, 128
_EPS = 1e-5
_BI = 32


def _layernorm(x, g, b):
    mu = jnp.mean(x, axis=-1, keepdims=True)
    var = jnp.mean((x - mu) ** 2, axis=-1, keepdims=True)
    return (x - mu) / jnp.sqrt(var + _EPS) * g + b


def _proj_kernel(hsc_ref, hst_ref, wsc_ref, wst_ref, bsc_ref, bst_ref,
                 osc_ref, ost_ref):
    hsc = hsc_ref[...].reshape(_B * _VSC, _H)
    hst = hst_ref[...].reshape(_B * _VST, _H)
    osc = jnp.dot(hsc, wsc_ref[...], preferred_element_type=jnp.float32)
    ost = jnp.dot(hst, wst_ref[...], preferred_element_type=jnp.float32)
    osc_ref[...] = (osc + bsc_ref[...]).reshape(_B, _VSC, 6 * _H)
    ost_ref[...] = (ost + bst_ref[...]).reshape(_B, _VST, 6 * _H)


def _edge_kernel(e_ref, arow_ref, bcol_ref, vrow_ref, wc_ref, avg_ref,
                 ge_ref, be_ref, *rest, ncols, with_col):
    if with_col:
        vcol_ref, eout_ref, row_ref, col_ref = rest
    else:
        eout_ref, row_ref = rest
    x = e_ref[0]
    xm = jnp.dot(x.reshape(_BI * ncols, _H), wc_ref[...],
                 preferred_element_type=jnp.float32)
    e_new = (xm.reshape(_BI, ncols, _H)
             + arow_ref[0][:, None, :] + bcol_ref[0][None, :, :])
    g = 1.0 / (1.0 + jnp.exp(-e_new))
    row_ref[0] = jnp.sum(g * vrow_ref[0][None, :, :], axis=1)
    if with_col:
        part = jnp.sum(g * vcol_ref[0][:, None, :], axis=0)

        @pl.when(pl.program_id(1) == 0)
        def _():
            col_ref[0] = part

        @pl.when(pl.program_id(1) != 0)
        def _():
            col_ref[0] += part

    e2 = e_new.reshape(_BI * ncols, _H)
    mu = jnp.dot(e2, avg_ref[...], preferred_element_type=jnp.float32)
    msq = jnp.dot(e2 * e2, avg_ref[...], preferred_element_type=jnp.float32)
    scale = jax.lax.rsqrt(msq - mu * mu + _EPS) * ge_ref[0]
    ln = (e2 - mu) * scale + be_ref[0]
    eout_ref[0] = x + jnp.maximum(ln, 0.0).reshape(_BI, ncols, _H)


def _node_kernel(uhsc_ref, uhst_ref, st2sc_ref, sc2sc_ref, sc2st_ref,
                 st2st_ref, hsc_ref, hst_ref, gh_ref, bh_ref,
                 osc_ref, ost_ref):
    xsc = uhsc_ref[...] + st2sc_ref[...] + sc2sc_ref[...]
    xst = uhst_ref[...] + sc2st_ref[...] + st2st_ref[...]
    osc_ref[...] = hsc_ref[...] + jnp.maximum(
        _layernorm(xsc, gh_ref[0], bh_ref[0]), 0.0)
    ost_ref[...] = hst_ref[...] + jnp.maximum(
        _layernorm(xst, gh_ref[0], bh_ref[0]), 0.0)


def _edge_call(e, proj_row, proj_st_or_sc, arow_idx, bcol_idx, vrow_idx,
               wc, ge, be, nrows, ncols, with_col, vcol_idx=None):
    nblk = nrows // _BI
    vec = lambda v: v.reshape(1, _H)
    small = pl.BlockSpec((1, _H), lambda b, i: (0, 0))
    full_col = lambda idx: pl.BlockSpec((1, ncols, _H),
                                        lambda b, i, idx=idx: (b, 0, idx))
    row_blk = lambda idx: pl.BlockSpec((1, _BI, _H),
                                       lambda b, i, idx=idx: (b, i, idx))
    in_specs = [
        pl.BlockSpec((1, _BI, ncols, _H), lambda b, i: (b, i, 0, 0)),
        row_blk(arow_idx),
        full_col(bcol_idx),
        full_col(vrow_idx),
        pl.BlockSpec((_H, _H), lambda b, i: (0, 0)),
        pl.BlockSpec((_H, _H), lambda b, i: (0, 0)),
        small, small,
    ]
    out_shapes = [
        jax.ShapeDtypeStruct((_B, nrows, ncols, _H), jnp.float32),
        jax.ShapeDtypeStruct((_B, nrows, _H), jnp.float32),
    ]
    out_specs = [
        pl.BlockSpec((1, _BI, ncols, _H), lambda b, i: (b, i, 0, 0)),
        pl.BlockSpec((1, _BI, _H), lambda b, i: (b, i, 0)),
    ]
    args = [e, proj_row, proj_st_or_sc, proj_st_or_sc, wc,
            jnp.full((_H, _H), 1.0 / _H, jnp.float32), vec(ge), vec(be)]
    if with_col:
        in_specs.append(row_blk(vcol_idx))
        args.append(proj_row)
        out_shapes.append(jax.ShapeDtypeStruct((_B, ncols, _H), jnp.float32))
        out_specs.append(pl.BlockSpec((1, ncols, _H), lambda b, i: (b, 0, 0)))
    return pl.pallas_call(
        functools.partial(_edge_kernel, ncols=ncols, with_col=with_col),
        grid=(_B, nblk),
        in_specs=in_specs,
        out_specs=out_specs,
        out_shape=out_shapes,
    )(*args)


def kernel(h_sc, h_st, bi_e, bi_graph, sc_e, sc_graph, st_e, st_graph,
           params):
    p = params
    wsc = jnp.concatenate([p["U1"]["w"], p["V1"]["w"], p["W1"]["w"],
                           p["bi_A"]["w"], p["sc_A"]["w"], p["sc_B"]["w"]],
                          axis=0).T
    wst = jnp.concatenate([p["U2"]["w"], p["V2"]["w"], p["W2"]["w"],
                           p["bi_B"]["w"], p["st_A"]["w"], p["st_B"]["w"]],
                          axis=0).T
    bsc = jnp.concatenate([p["U1"]["b"], p["V1"]["b"], p["W1"]["b"],
                           p["bi_A"]["b"] + p["bi_C"]["b"],
                           p["sc_A"]["b"] + p["sc_C"]["b"],
                           p["sc_B"]["b"]]).reshape(1, 6 * _H)
    bst = jnp.concatenate([p["U2"]["b"], p["V2"]["b"], p["W2"]["b"],
                           p["bi_B"]["b"],
                           p["st_A"]["b"] + p["st_C"]["b"],
                           p["st_B"]["b"]]).reshape(1, 6 * _H)

    proj_sc, proj_st = pl.pallas_call(
        _proj_kernel,
        out_shape=[jax.ShapeDtypeStruct((_B, _VSC, 6 * _H), jnp.float32),
                   jax.ShapeDtypeStruct((_B, _VST, 6 * _H), jnp.float32)],
    )(h_sc, h_st, wsc, wst, bsc, bst)

    ge, be = p["ln_e"]["g"], p["ln_e"]["b"]

    bi_e_out, h_st2sc, h_sc2st = _edge_call(
        bi_e, proj_sc, proj_st, arow_idx=3, bcol_idx=3, vrow_idx=1,
        wc=p["bi_C"]["w"].T, ge=ge, be=be,
        nrows=_VSC, ncols=_VST, with_col=True, vcol_idx=1)
    sc_e_out, h_sc2sc = _edge_call(
        sc_e, proj_sc, proj_sc, arow_idx=4, bcol_idx=5, vrow_idx=2,
        wc=p["sc_C"]["w"].T, ge=ge, be=be,
        nrows=_VSC, ncols=_VSC, with_col=False)
    st_e_out, h_st2st = _edge_call(
        st_e, proj_st, proj_st, arow_idx=4, bcol_idx=5, vrow_idx=2,
        wc=p["st_C"]["w"].T, ge=ge, be=be,
        nrows=_VST, ncols=_VST, with_col=False)

    full = pl.BlockSpec((_B, _VSC, _H), lambda i: (0, 0, 0))
    small = pl.BlockSpec((1, _H), lambda i: (0, 0))
    h_sc_out, h_st_out = pl.pallas_call(
        _node_kernel,
        grid=(1,),
        in_specs=[full, full, full, full, full, full, full, full,
                  small, small],
        out_specs=[full, full],
        out_shape=[jax.ShapeDtypeStruct((_B, _VSC, _H), jnp.float32),
                   jax.ShapeDtypeStruct((_B, _VST, _H), jnp.float32)],
    )(proj_sc, proj_st, h_st2sc, h_sc2sc, h_sc2st, h_st2st, h_sc, h_st,
      p["ln_h"]["g"].reshape(1, _H), p["ln_h"]["b"].reshape(1, _H))

    return (h_sc_out, h_st_out, bi_e_out, sc_e_out, st_e_out)


# speedup vs baseline: 2.4786x; 1.0257x over previous
import functools

import jax
import jax.numpy as jnp
from jax.experimental import pallas as pl
from jax.experimental.pallas import tpu as pltpu

_B, _VSC, _VST, _H = 2, 256, 256, 128
_EPS = 1e-5
_BI = 32
_NBLK = _VSC // _BI


def _ln_relu_mxu(x2, avg, g, b):
    mu = jnp.dot(x2, avg, preferred_element_type=jnp.float32)
    msq = jnp.dot(x2 * x2, avg, preferred_element_type=jnp.float32)
    scale = jax.lax.rsqrt(msq - mu * mu + _EPS) * g
    return jnp.maximum((x2 - mu) * scale + b, 0.0)


def _edge_body(e_ref, wc_ref, avg_ref, ge_ref, be_ref, eout_ref,
               arow, bcol, vrow, ncols):
    x = e_ref[0]
    xm = jnp.dot(x.reshape(_BI * ncols, _H), wc_ref[...],
                 preferred_element_type=jnp.float32)
    e_new = (xm.reshape(_BI, ncols, _H)
             + arow[:, None, :] + bcol[None, :, :])
    g = 1.0 / (1.0 + jnp.exp(-e_new))
    row = jnp.sum(g * vrow[None, :, :], axis=1)
    ln = _ln_relu_mxu(e_new.reshape(_BI * ncols, _H), avg_ref[...],
                      ge_ref[0], be_ref[0])
    eout_ref[0] = x + ln.reshape(_BI, ncols, _H)
    return g, row


def _bi_kernel(e_ref, hsc_ref, hst_ref, wrow_ref, brow_ref, wcol_ref,
               bcol_ref, wc_ref, avg_ref, ge_ref, be_ref,
               eout_ref, row_ref, col_ref, prow_ref, pcol_ref):
    b, i = pl.program_id(0), pl.program_id(1)

    @pl.when((b == 0) & (i == 0))
    def _():
        hsc = hsc_ref[...].reshape(_B * _VSC, _H)
        hst = hst_ref[...].reshape(_B * _VST, _H)
        prow_ref[...] = jnp.dot(hsc, wrow_ref[...],
                                preferred_element_type=jnp.float32) + brow_ref[...]
        pcol_ref[...] = jnp.dot(hst, wcol_ref[...],
                                preferred_element_type=jnp.float32) + bcol_ref[...]

    base = b * _VSC + i * _BI
    arow = prow_ref[pl.ds(base, _BI), 0:_H]
    vcol = prow_ref[pl.ds(base, _BI), _H:2 * _H]
    bcol = pcol_ref[pl.ds(b * _VST, _VST), 0:_H]
    vrow = pcol_ref[pl.ds(b * _VST, _VST), _H:2 * _H]

    g, row = _edge_body(e_ref, wc_ref, avg_ref, ge_ref, be_ref, eout_ref,
                        arow, bcol, vrow, _VST)
    row_ref[0] = row
    part = jnp.sum(g * vcol[:, None, :], axis=0)

    @pl.when(i == 0)
    def _():
        col_ref[0] = part

    @pl.when(i != 0)
    def _():
        col_ref[0] += part


def _sq_kernel(e_ref, h_ref, wp_ref, bp_ref, wc_ref, avg_ref, ge_ref,
               be_ref, eout_ref, row_ref, p_ref, *, nv):
    b, i = pl.program_id(0), pl.program_id(1)

    @pl.when((b == 0) & (i == 0))
    def _():
        h = h_ref[...].reshape(_B * nv, _H)
        p_ref[...] = jnp.dot(h, wp_ref[...],
                             preferred_element_type=jnp.float32) + bp_ref[...]

    base = b * nv + i * _BI
    arow = p_ref[pl.ds(base, _BI), 0:_H]
    bcol = p_ref[pl.ds(b * nv, nv), _H:2 * _H]
    vrow = p_ref[pl.ds(b * nv, nv), 2 * _H:3 * _H]

    _, row = _edge_body(e_ref, wc_ref, avg_ref, ge_ref, be_ref, eout_ref,
                        arow, bcol, vrow, nv)
    row_ref[0] = row


def _st_kernel(e_ref, hst_ref, wp_ref, bp_ref, wc_ref, avg_ref, ge_ref,
               be_ref, hsc_ref, wu1_ref, wu2_ref, bu_ref, st2sc_ref,
               sc2sc_ref, sc2st_ref, gh_ref, bh_ref,
               eout_ref, hsc_out_ref, hst_out_ref, p_ref, agg_ref):
    b, i = pl.program_id(0), pl.program_id(1)

    @pl.when((b == 0) & (i == 0))
    def _():
        h = hst_ref[...].reshape(_B * _VST, _H)
        p_ref[...] = jnp.dot(h, wp_ref[...],
                             preferred_element_type=jnp.float32) + bp_ref[...]

    base = b * _VST + i * _BI
    arow = p_ref[pl.ds(base, _BI), 0:_H]
    bcol = p_ref[pl.ds(b * _VST, _VST), _H:2 * _H]
    vrow = p_ref[pl.ds(b * _VST, _VST), 2 * _H:3 * _H]

    _, row = _edge_body(e_ref, wc_ref, avg_ref, ge_ref, be_ref, eout_ref,
                        arow, bcol, vrow, _VST)
    agg_ref[pl.ds(base, _BI), :] = row

    @pl.when((b == _B - 1) & (i == _NBLK - 1))
    def _():
        hsc = hsc_ref[...].reshape(_B * _VSC, _H)
        hst = hst_ref[...].reshape(_B * _VST, _H)
        uhsc = jnp.dot(hsc, wu1_ref[...],
                       preferred_element_type=jnp.float32) + bu_ref[0, 0:_H]
        uhst = jnp.dot(hst, wu2_ref[...],
                       preferred_element_type=jnp.float32) + bu_ref[0, _H:2 * _H]
        xsc = (uhsc + st2sc_ref[...].reshape(_B * _VSC, _H)
               + sc2sc_ref[...].reshape(_B * _VSC, _H))
        xst = (uhst + sc2st_ref[...].reshape(_B * _VST, _H)
               + agg_ref[...])
        osc = hsc + _ln_relu_mxu(xsc, avg_ref[...], gh_ref[0], bh_ref[0])
        ost = hst + _ln_relu_mxu(xst, avg_ref[...], gh_ref[0], bh_ref[0])
        hsc_out_ref[...] = osc.reshape(_B, _VSC, _H)
        hst_out_ref[...] = ost.reshape(_B, _VST, _H)


def kernel(h_sc, h_st, bi_e, bi_graph, sc_e, sc_graph, st_e, st_graph,
           params):
    p = params
    f32 = jnp.float32
    avg = jnp.full((_H, _H), 1.0 / _H, f32)
    vec = lambda v: v.reshape(1, -1)
    ge, be = vec(p["ln_e"]["g"]), vec(p["ln_e"]["b"])
    gh, bh = vec(p["ln_h"]["g"]), vec(p["ln_h"]["b"])

    w_bi_row = jnp.concatenate([p["bi_A"]["w"], p["V1"]["w"]], axis=0).T
    b_bi_row = vec(jnp.concatenate([p["bi_A"]["b"] + p["bi_C"]["b"],
                                    p["V1"]["b"]]))
    w_bi_col = jnp.concatenate([p["bi_B"]["w"], p["V2"]["w"]], axis=0).T
    b_bi_col = vec(jnp.concatenate([p["bi_B"]["b"], p["V2"]["b"]]))
    w_sc = jnp.concatenate([p["sc_A"]["w"], p["sc_B"]["w"],
                            p["W1"]["w"]], axis=0).T
    b_sc = vec(jnp.concatenate([p["sc_A"]["b"] + p["sc_C"]["b"],
                                p["sc_B"]["b"], p["W1"]["b"]]))
    w_st = jnp.concatenate([p["st_A"]["w"], p["st_B"]["w"],
                            p["W2"]["w"]], axis=0).T
    b_st = vec(jnp.concatenate([p["st_A"]["b"] + p["st_C"]["b"],
                                p["st_B"]["b"], p["W2"]["b"]]))
    b_u = vec(jnp.concatenate([p["U1"]["b"], p["U2"]["b"]]))

    const = lambda shape: pl.BlockSpec(shape, lambda b, i: (0,) * len(shape))
    eblk = lambda ncols: pl.BlockSpec((1, _BI, ncols, _H),
                                      lambda b, i: (b, i, 0, 0))
    rowblk = pl.BlockSpec((1, _BI, _H), lambda b, i: (b, i, 0))
    nodes = lambda nv: const((_B, nv, _H))

    bi_e_out, h_st2sc, h_sc2st = pl.pallas_call(
        _bi_kernel,
        grid=(_B, _NBLK),
        in_specs=[eblk(_VST), nodes(_VSC), nodes(_VST),
                  const((_H, 2 * _H)), const((1, 2 * _H)),
                  const((_H, 2 * _H)), const((1, 2 * _H)),
                  const((_H, _H)), const((_H, _H)),
                  const((1, _H)), const((1, _H))],
        out_specs=[eblk(_VST), rowblk,
                   pl.BlockSpec((1, _VST, _H), lambda b, i: (b, 0, 0))],
        out_shape=[jax.ShapeDtypeStruct((_B, _VSC, _VST, _H), f32),
                   jax.ShapeDtypeStruct((_B, _VSC, _H), f32),
                   jax.ShapeDtypeStruct((_B, _VST, _H), f32)],
        scratch_shapes=[pltpu.VMEM((_B * _VSC, 2 * _H), f32),
                        pltpu.VMEM((_B * _VST, 2 * _H), f32)],
    )(bi_e, h_sc, h_st, w_bi_row, b_bi_row, w_bi_col, b_bi_col,
      p["bi_C"]["w"].T, avg, ge, be)

    sc_e_out, h_sc2sc = pl.pallas_call(
        functools.partial(_sq_kernel, nv=_VSC),
        grid=(_B, _NBLK),
        in_specs=[eblk(_VSC), nodes(_VSC),
                  const((_H, 3 * _H)), const((1, 3 * _H)),
                  const((_H, _H)), const((_H, _H)),
                  const((1, _H)), const((1, _H))],
        out_specs=[eblk(_VSC), rowblk],
        out_shape=[jax.ShapeDtypeStruct((_B, _VSC, _VSC, _H), f32),
                   jax.ShapeDtypeStruct((_B, _VSC, _H), f32)],
        scratch_shapes=[pltpu.VMEM((_B * _VSC, 3 * _H), f32)],
    )(sc_e, h_sc, w_sc, b_sc, p["sc_C"]["w"].T, avg, ge, be)

    st_e_out, h_sc_out, h_st_out = pl.pallas_call(
        _st_kernel,
        grid=(_B, _NBLK),
        in_specs=[eblk(_VST), nodes(_VST),
                  const((_H, 3 * _H)), const((1, 3 * _H)),
                  const((_H, _H)), const((_H, _H)),
                  const((1, _H)), const((1, _H)),
                  nodes(_VSC), const((_H, _H)), const((_H, _H)),
                  const((1, 2 * _H)), nodes(_VSC), nodes(_VSC),
                  nodes(_VST), const((1, _H)), const((1, _H))],
        out_specs=[eblk(_VST), nodes(_VSC), nodes(_VST)],
        out_shape=[jax.ShapeDtypeStruct((_B, _VST, _VST, _H), f32),
                   jax.ShapeDtypeStruct((_B, _VSC, _H), f32),
                   jax.ShapeDtypeStruct((_B, _VST, _H), f32)],
        scratch_shapes=[pltpu.VMEM((_B * _VST, 3 * _H), f32),
                        pltpu.VMEM((_B * _VST, _H), f32)],
    )(st_e, h_st, w_st, b_st, p["st_C"]["w"].T, avg, ge, be,
      h_sc, p["U1"]["w"].T, p["U2"]["w"].T, b_u,
      h_st2sc, h_sc2sc, h_sc2st, gh, bh)

    return (h_sc_out, h_st_out, bi_e_out, sc_e_out, st_e_out)


# speedup vs baseline: 2.5868x; 1.0437x over previous
import functools

import jax
import jax.numpy as jnp
from jax.experimental import pallas as pl
from jax.experimental.pallas import tpu as pltpu

_B, _VSC, _VST, _H = 2, 256, 256, 128
_EPS = 1e-5
_BI = 64
_NBLK = _VSC // _BI


def _ln_relu_mxu(x2, avg, g, b):
    mu = jnp.dot(x2, avg, preferred_element_type=jnp.float32)
    msq = jnp.dot(x2 * x2, avg, preferred_element_type=jnp.float32)
    scale = jax.lax.rsqrt(msq - mu * mu + _EPS) * g
    return jnp.maximum((x2 - mu) * scale + b, 0.0)


def _edge_body(e_ref, wc_ref, avg_ref, ge_ref, be_ref, eout_ref,
               arow, bcol, vrow, ncols):
    x = e_ref[0]
    xm = jnp.dot(x.reshape(_BI * ncols, _H), wc_ref[...],
                 preferred_element_type=jnp.float32)
    e_new = (xm.reshape(_BI, ncols, _H)
             + arow[:, None, :] + bcol[None, :, :])
    g = 1.0 / (1.0 + jnp.exp(-e_new))
    row = jnp.sum(g * vrow[None, :, :], axis=1)
    ln = _ln_relu_mxu(e_new.reshape(_BI * ncols, _H), avg_ref[...],
                      ge_ref[0], be_ref[0])
    eout_ref[0] = x + ln.reshape(_BI, ncols, _H)
    return g, row


def _bi_kernel(e_ref, hsc_ref, hst_ref, wrow_ref, brow_ref, wcol_ref,
               bcol_ref, wc_ref, avg_ref, ge_ref, be_ref,
               eout_ref, row_ref, col_ref, prow_ref, pcol_ref):
    b, i = pl.program_id(0), pl.program_id(1)

    @pl.when((b == 0) & (i == 0))
    def _():
        hsc = hsc_ref[...].reshape(_B * _VSC, _H)
        hst = hst_ref[...].reshape(_B * _VST, _H)
        prow_ref[...] = jnp.dot(hsc, wrow_ref[...],
                                preferred_element_type=jnp.float32) + brow_ref[...]
        pcol_ref[...] = jnp.dot(hst, wcol_ref[...],
                                preferred_element_type=jnp.float32) + bcol_ref[...]

    base = b * _VSC + i * _BI
    arow = prow_ref[pl.ds(base, _BI), 0:_H]
    vcol = prow_ref[pl.ds(base, _BI), _H:2 * _H]
    bcol = pcol_ref[pl.ds(b * _VST, _VST), 0:_H]
    vrow = pcol_ref[pl.ds(b * _VST, _VST), _H:2 * _H]

    g, row = _edge_body(e_ref, wc_ref, avg_ref, ge_ref, be_ref, eout_ref,
                        arow, bcol, vrow, _VST)
    row_ref[0] = row
    part = jnp.sum(g * vcol[:, None, :], axis=0)

    @pl.when(i == 0)
    def _():
        col_ref[0] = part

    @pl.when(i != 0)
    def _():
        col_ref[0] += part


def _sq_kernel(e_ref, h_ref, wp_ref, bp_ref, wc_ref, avg_ref, ge_ref,
               be_ref, eout_ref, row_ref, p_ref, *, nv):
    b, i = pl.program_id(0), pl.program_id(1)

    @pl.when((b == 0) & (i == 0))
    def _():
        h = h_ref[...].reshape(_B * nv, _H)
        p_ref[...] = jnp.dot(h, wp_ref[...],
                             preferred_element_type=jnp.float32) + bp_ref[...]

    base = b * nv + i * _BI
    arow = p_ref[pl.ds(base, _BI), 0:_H]
    bcol = p_ref[pl.ds(b * nv, nv), _H:2 * _H]
    vrow = p_ref[pl.ds(b * nv, nv), 2 * _H:3 * _H]

    _, row = _edge_body(e_ref, wc_ref, avg_ref, ge_ref, be_ref, eout_ref,
                        arow, bcol, vrow, nv)
    row_ref[0] = row


def _st_kernel(e_ref, hst_ref, wp_ref, bp_ref, wc_ref, avg_ref, ge_ref,
               be_ref, hsc_ref, wu1_ref, wu2_ref, bu_ref, st2sc_ref,
               sc2sc_ref, sc2st_ref, gh_ref, bh_ref,
               eout_ref, hsc_out_ref, hst_out_ref, p_ref, agg_ref):
    b, i = pl.program_id(0), pl.program_id(1)

    @pl.when((b == 0) & (i == 0))
    def _():
        h = hst_ref[...].reshape(_B * _VST, _H)
        p_ref[...] = jnp.dot(h, wp_ref[...],
                             preferred_element_type=jnp.float32) + bp_ref[...]

    base = b * _VST + i * _BI
    arow = p_ref[pl.ds(base, _BI), 0:_H]
    bcol = p_ref[pl.ds(b * _VST, _VST), _H:2 * _H]
    vrow = p_ref[pl.ds(b * _VST, _VST), 2 * _H:3 * _H]

    _, row = _edge_body(e_ref, wc_ref, avg_ref, ge_ref, be_ref, eout_ref,
                        arow, bcol, vrow, _VST)
    agg_ref[pl.ds(base, _BI), :] = row

    @pl.when((b == _B - 1) & (i == _NBLK - 1))
    def _():
        hsc = hsc_ref[...].reshape(_B * _VSC, _H)
        hst = hst_ref[...].reshape(_B * _VST, _H)
        uhsc = jnp.dot(hsc, wu1_ref[...],
                       preferred_element_type=jnp.float32) + bu_ref[0, 0:_H]
        uhst = jnp.dot(hst, wu2_ref[...],
                       preferred_element_type=jnp.float32) + bu_ref[0, _H:2 * _H]
        xsc = (uhsc + st2sc_ref[...].reshape(_B * _VSC, _H)
               + sc2sc_ref[...].reshape(_B * _VSC, _H))
        xst = (uhst + sc2st_ref[...].reshape(_B * _VST, _H)
               + agg_ref[...])
        osc = hsc + _ln_relu_mxu(xsc, avg_ref[...], gh_ref[0], bh_ref[0])
        ost = hst + _ln_relu_mxu(xst, avg_ref[...], gh_ref[0], bh_ref[0])
        hsc_out_ref[...] = osc.reshape(_B, _VSC, _H)
        hst_out_ref[...] = ost.reshape(_B, _VST, _H)


def kernel(h_sc, h_st, bi_e, bi_graph, sc_e, sc_graph, st_e, st_graph,
           params):
    p = params
    f32 = jnp.float32
    avg = jnp.full((_H, _H), 1.0 / _H, f32)
    vec = lambda v: v.reshape(1, -1)
    ge, be = vec(p["ln_e"]["g"]), vec(p["ln_e"]["b"])
    gh, bh = vec(p["ln_h"]["g"]), vec(p["ln_h"]["b"])

    w_bi_row = jnp.concatenate([p["bi_A"]["w"], p["V1"]["w"]], axis=0).T
    b_bi_row = vec(jnp.concatenate([p["bi_A"]["b"] + p["bi_C"]["b"],
                                    p["V1"]["b"]]))
    w_bi_col = jnp.concatenate([p["bi_B"]["w"], p["V2"]["w"]], axis=0).T
    b_bi_col = vec(jnp.concatenate([p["bi_B"]["b"], p["V2"]["b"]]))
    w_sc = jnp.concatenate([p["sc_A"]["w"], p["sc_B"]["w"],
                            p["W1"]["w"]], axis=0).T
    b_sc = vec(jnp.concatenate([p["sc_A"]["b"] + p["sc_C"]["b"],
                                p["sc_B"]["b"], p["W1"]["b"]]))
    w_st = jnp.concatenate([p["st_A"]["w"], p["st_B"]["w"],
                            p["W2"]["w"]], axis=0).T
    b_st = vec(jnp.concatenate([p["st_A"]["b"] + p["st_C"]["b"],
                                p["st_B"]["b"], p["W2"]["b"]]))
    b_u = vec(jnp.concatenate([p["U1"]["b"], p["U2"]["b"]]))

    const = lambda shape: pl.BlockSpec(shape, lambda b, i: (0,) * len(shape))
    eblk = lambda ncols: pl.BlockSpec((1, _BI, ncols, _H),
                                      lambda b, i: (b, i, 0, 0))
    rowblk = pl.BlockSpec((1, _BI, _H), lambda b, i: (b, i, 0))
    nodes = lambda nv: const((_B, nv, _H))

    bi_e_out, h_st2sc, h_sc2st = pl.pallas_call(
        _bi_kernel,
        grid=(_B, _NBLK),
        in_specs=[eblk(_VST), nodes(_VSC), nodes(_VST),
                  const((_H, 2 * _H)), const((1, 2 * _H)),
                  const((_H, 2 * _H)), const((1, 2 * _H)),
                  const((_H, _H)), const((_H, _H)),
                  const((1, _H)), const((1, _H))],
        out_specs=[eblk(_VST), rowblk,
                   pl.BlockSpec((1, _VST, _H), lambda b, i: (b, 0, 0))],
        out_shape=[jax.ShapeDtypeStruct((_B, _VSC, _VST, _H), f32),
                   jax.ShapeDtypeStruct((_B, _VSC, _H), f32),
                   jax.ShapeDtypeStruct((_B, _VST, _H), f32)],
        scratch_shapes=[pltpu.VMEM((_B * _VSC, 2 * _H), f32),
                        pltpu.VMEM((_B * _VST, 2 * _H), f32)],
    )(bi_e, h_sc, h_st, w_bi_row, b_bi_row, w_bi_col, b_bi_col,
      p["bi_C"]["w"].T, avg, ge, be)

    sc_e_out, h_sc2sc = pl.pallas_call(
        functools.partial(_sq_kernel, nv=_VSC),
        grid=(_B, _NBLK),
        in_specs=[eblk(_VSC), nodes(_VSC),
                  const((_H, 3 * _H)), const((1, 3 * _H)),
                  const((_H, _H)), const((_H, _H)),
                  const((1, _H)), const((1, _H))],
        out_specs=[eblk(_VSC), rowblk],
        out_shape=[jax.ShapeDtypeStruct((_B, _VSC, _VSC, _H), f32),
                   jax.ShapeDtypeStruct((_B, _VSC, _H), f32)],
        scratch_shapes=[pltpu.VMEM((_B * _VSC, 3 * _H), f32)],
    )(sc_e, h_sc, w_sc, b_sc, p["sc_C"]["w"].T, avg, ge, be)

    st_e_out, h_sc_out, h_st_out = pl.pallas_call(
        _st_kernel,
        grid=(_B, _NBLK),
        in_specs=[eblk(_VST), nodes(_VST),
                  const((_H, 3 * _H)), const((1, 3 * _H)),
                  const((_H, _H)), const((_H, _H)),
                  const((1, _H)), const((1, _H)),
                  nodes(_VSC), const((_H, _H)), const((_H, _H)),
                  const((1, 2 * _H)), nodes(_VSC), nodes(_VSC),
                  nodes(_VST), const((1, _H)), const((1, _H))],
        out_specs=[eblk(_VST), nodes(_VSC), nodes(_VST)],
        out_shape=[jax.ShapeDtypeStruct((_B, _VST, _VST, _H), f32),
                   jax.ShapeDtypeStruct((_B, _VSC, _H), f32),
                   jax.ShapeDtypeStruct((_B, _VST, _H), f32)],
        scratch_shapes=[pltpu.VMEM((_B * _VST, 3 * _H), f32),
                        pltpu.VMEM((_B * _VST, _H), f32)],
    )(st_e, h_st, w_st, b_st, p["st_C"]["w"].T, avg, ge, be,
      h_sc, p["U1"]["w"].T, p["U2"]["w"].T, b_u,
      h_st2sc, h_sc2sc, h_sc2st, gh, bh)

    return (h_sc_out, h_st_out, bi_e_out, sc_e_out, st_e_out)
